# Initial kernel scaffold; baseline (speedup 1.0000x reference)
import jax
import jax.numpy as jnp
from jax.experimental import pallas as pl


def kernel(memory, source_nodes, timestamps, n_layers, neighbors, edge_idxs, edge_times, node_features, edge_features, time_w, time_b, Wq, Wk, Wv, Wout):
    raise NotImplementedError("write your pallas kernel here")


# speedup vs baseline: 2.5053x; 2.5053x over previous
import functools

import jax
import jax.numpy as jnp
import numpy as np
from jax import lax
from jax.experimental import pallas as pl
from jax.experimental.pallas import tpu as pltpu
from jax.experimental.pallas import tpu_sc as plsc

N_NODES = 100000
N_EDGES = 1600000
B = 2048
K = 20
D = 128
D_TIME = 128
D_EDGE = 16
N_HEADS = 2
DH = D // N_HEADS

NC = 2
NS = 16
NW = NC * NS
BK = B * K
ROWS_W = BK // NW
SRC_W = B // NW
CH = 128
NCH = ROWS_W // CH


def _sc_gather_body(mem_hbm, nf_hbm, sidx_hbm, nidx_hbm,
                    src_out, nbr_out,
                    sidx_v, nidx_v, a_v, b_v, sa_v, sb_v,
                    sem_a, sem_b, sem_w):
    wid = lax.axis_index("s") * NC + lax.axis_index("c")
    nbase = wid * ROWS_W

    pltpu.sync_copy(nidx_hbm.at[wid], nidx_v)
    pltpu.sync_copy(sidx_hbm.at[pl.ds(wid * SRC_W, SRC_W)], sidx_v)

    cp_a = pltpu.async_copy(mem_hbm.at[sidx_v], sa_v, sem_a)
    cp_b = pltpu.async_copy(nf_hbm.at[sidx_v], sb_v, sem_b)
    cp_a.wait()
    cp_b.wait()

    def _add_src(r, carry):
        for s8 in range(D // 16):
            sl = pl.ds(s8 * 16, 16)
            sa_v[r, sl] = sa_v[r, sl] + sb_v[r, sl]
        return carry

    lax.fori_loop(0, SRC_W, _add_src, 0)
    pltpu.async_copy(sa_v, src_out.at[pl.ds(wid * SRC_W, SRC_W)], sem_w).wait()

    def _add_nbr(r, carry):
        for s8 in range(D // 16):
            sl = pl.ds(s8 * 16, 16)
            a_v[r, sl] = a_v[r, sl] + b_v[r, sl]
        return carry

    for c in range(NCH):
        cp_a = pltpu.async_copy(mem_hbm.at[nidx_v.at[c]], a_v, sem_a)
        cp_b = pltpu.async_copy(nf_hbm.at[nidx_v.at[c]], b_v, sem_b)
        cp_a.wait()
        cp_b.wait()
        lax.fori_loop(0, CH, _add_nbr, 0)
        pltpu.async_copy(a_v, nbr_out.at[pl.ds(nbase + c * CH, CH)], sem_w).wait()


def _sc_gather(memory, node_features, src_idx, nbr_idx3):
    mesh = plsc.VectorSubcoreMesh(core_axis_name="c", subcore_axis_name="s")
    fn = pl.kernel(
        _sc_gather_body,
        mesh=mesh,
        out_type=(
            jax.ShapeDtypeStruct((B, D), jnp.float32),
            jax.ShapeDtypeStruct((BK, D), jnp.float32),
        ),
        scratch_types=[
            pltpu.VMEM((SRC_W,), jnp.int32),
            pltpu.VMEM((NCH, CH), jnp.int32),
            pltpu.VMEM((CH, D), jnp.float32),
            pltpu.VMEM((CH, D), jnp.float32),
            pltpu.VMEM((SRC_W, D), jnp.float32),
            pltpu.VMEM((SRC_W, D), jnp.float32),
            pltpu.SemaphoreType.DMA,
            pltpu.SemaphoreType.DMA,
            pltpu.SemaphoreType.DMA,
        ],
    )
    return fn(memory, node_features, src_idx, nbr_idx3)


BB = 256
_PREC = lax.Precision.HIGHEST


def _tc_body(src_ref, nbr_ref, ef_ref, ts_ref, ets_ref, nid_ref, tw_ref, tb_ref,
             wq1, wq2, wk1, wk2, wk3, wv1, wv2, wv3, wo1, wo2, out_ref):
    src = src_ref[...]
    tw = tw_ref[...]
    tb = tb_ref[...]
    q_const = jnp.dot(jnp.cos(tb), wq2[...], precision=_PREC)
    q = jnp.dot(src, wq1[...], precision=_PREC) + q_const
    scale = np.float32(1.0 / np.sqrt(DH))

    neg = jnp.full((BB, 1), -1e30, dtype=jnp.float32)
    m0, m1 = neg, neg
    l0 = jnp.zeros((BB, 1), dtype=jnp.float32)
    l1 = jnp.zeros((BB, 1), dtype=jnp.float32)
    acc0 = jnp.zeros((BB, DH), dtype=jnp.float32)
    acc1 = jnp.zeros((BB, DH), dtype=jnp.float32)

    for k in range(K):
        nbr_k = nbr_ref[k]
        ef_k = ef_ref[:, k * D_EDGE:(k + 1) * D_EDGE]
        delta = ts_ref[...] - ets_ref[:, k:k + 1]
        te = jnp.cos(delta * tw + tb)
        kk = (jnp.dot(nbr_k, wk1[...], precision=_PREC)
              + jnp.dot(te, wk2[...], precision=_PREC)
              + jnp.dot(ef_k, wk3[...], precision=_PREC))
        vv = (jnp.dot(nbr_k, wv1[...], precision=_PREC)
              + jnp.dot(te, wv2[...], precision=_PREC)
              + jnp.dot(ef_k, wv3[...], precision=_PREC))
        prod = q * kk
        s0 = jnp.sum(prod[:, :DH], axis=1, keepdims=True) * scale
        s1 = jnp.sum(prod[:, DH:], axis=1, keepdims=True) * scale
        is_pad = nid_ref[:, k:k + 1] == 0
        s0 = jnp.where(is_pad, jnp.float32(-1e9), s0)
        s1 = jnp.where(is_pad, jnp.float32(-1e9), s1)
        m0n = jnp.maximum(m0, s0)
        m1n = jnp.maximum(m1, s1)
        a0 = jnp.exp(m0 - m0n)
        a1 = jnp.exp(m1 - m1n)
        p0 = jnp.exp(s0 - m0n)
        p1 = jnp.exp(s1 - m1n)
        l0 = l0 * a0 + p0
        l1 = l1 * a1 + p1
        acc0 = acc0 * a0 + p0 * vv[:, :DH]
        acc1 = acc1 * a1 + p1 * vv[:, DH:]
        m0, m1 = m0n, m1n

    attn = jnp.concatenate([acc0 / l0, acc1 / l1], axis=1)
    out_ref[...] = (jnp.dot(attn, wo1[...], precision=_PREC)
                    + jnp.dot(src, wo2[...], precision=_PREC))


def _tc_dense(src_conv, nbr3, ef2, ts2, ets, nids, time_w, time_b2,
              Wq1, Wq2, Wk1, Wk2, Wk3, Wv1, Wv2, Wv3, Wo1, Wo2):
    row = lambda i: (i, 0)
    fixed = lambda i: (0, 0)
    return pl.pallas_call(
        _tc_body,
        grid=(B // BB,),
        in_specs=[
            pl.BlockSpec((BB, D), row),
            pl.BlockSpec((K, BB, D), lambda i: (0, i, 0)),
            pl.BlockSpec((BB, K * D_EDGE), row),
            pl.BlockSpec((BB, 1), row),
            pl.BlockSpec((BB, K), row),
            pl.BlockSpec((BB, K), row),
            pl.BlockSpec((1, D_TIME), fixed),
            pl.BlockSpec((1, D_TIME), fixed),
            pl.BlockSpec((D, D), fixed),
            pl.BlockSpec((D_TIME, D), fixed),
            pl.BlockSpec((D, D), fixed),
            pl.BlockSpec((D_TIME, D), fixed),
            pl.BlockSpec((D_EDGE, D), fixed),
            pl.BlockSpec((D, D), fixed),
            pl.BlockSpec((D_TIME, D), fixed),
            pl.BlockSpec((D_EDGE, D), fixed),
            pl.BlockSpec((D, D), fixed),
            pl.BlockSpec((D, D), fixed),
        ],
        out_specs=pl.BlockSpec((BB, D), row),
        out_shape=jax.ShapeDtypeStruct((B, D), jnp.float32),
    )(src_conv, nbr3, ef2, ts2, ets, nids, time_w, time_b2,
      Wq1, Wq2, Wk1, Wk2, Wk3, Wv1, Wv2, Wv3, Wo1, Wo2)


def kernel(memory, source_nodes, timestamps, n_layers, neighbors, edge_idxs,
           edge_times, node_features, edge_features, time_w, time_b,
           Wq, Wk, Wv, Wout):
    del n_layers
    src_idx = source_nodes.astype(jnp.int32)
    nbr_idx3 = neighbors.astype(jnp.int32).T.reshape(NW, NCH, CH)

    src_conv, nbr_emb = _sc_gather(memory, node_features, src_idx, nbr_idx3)
    nbr3 = nbr_emb.reshape(K, B, D)

    ef2 = jnp.take(edge_features, edge_idxs.reshape(-1), axis=0).reshape(B, K * D_EDGE)

    ts2 = timestamps.reshape(B, 1)
    time_b2 = time_b.reshape(1, D_TIME)

    Wq1, Wq2 = Wq[:D], Wq[D:]
    Wk1, Wk2, Wk3 = Wk[:D], Wk[D:D + D_TIME], Wk[D + D_TIME:]
    Wv1, Wv2, Wv3 = Wv[:D], Wv[D:D + D_TIME], Wv[D + D_TIME:]
    Wo1, Wo2 = Wout[:D], Wout[D:]

    return _tc_dense(src_conv, nbr3, ef2, ts2, edge_times, neighbors.astype(jnp.int32),
                     time_w, time_b2, Wq1, Wq2, Wk1, Wk2, Wk3, Wv1, Wv2, Wv3, Wo1, Wo2)


# speedup vs baseline: 3.5544x; 1.4188x over previous
import functools

import jax
import jax.numpy as jnp
import numpy as np
from jax import lax
from jax.experimental import pallas as pl
from jax.experimental.pallas import tpu as pltpu
from jax.experimental.pallas import tpu_sc as plsc

N_NODES = 100000
N_EDGES = 1600000
B = 2048
K = 20
D = 128
D_TIME = 128
D_EDGE = 16
N_HEADS = 2
DH = D // N_HEADS

NC = 2
NS = 16
NW = NC * NS
BK = B * K
ROWS_W = BK // NW
SRC_W = B // NW
CH = 128
NCH = ROWS_W // CH


def _sc_gather_body(mem_hbm, nf_hbm, sidx_hbm, nidx_hbm,
                    src_out, nbr_out,
                    sidx_v, nidx_v, a_v, b_v, sa_v, sb_v,
                    sem_a, sem_b, sem_w):
    wid = lax.axis_index("s") * NC + lax.axis_index("c")
    nbase = wid * ROWS_W

    pltpu.sync_copy(nidx_hbm.at[wid], nidx_v)
    pltpu.sync_copy(sidx_hbm.at[pl.ds(wid * SRC_W, SRC_W)], sidx_v)

    cp_a = pltpu.async_copy(mem_hbm.at[sidx_v], sa_v, sem_a)
    cp_b = pltpu.async_copy(nf_hbm.at[sidx_v], sb_v, sem_b)
    cp_a.wait()
    cp_b.wait()

    def _add_src(r, carry):
        for s8 in range(D // 16):
            sl = pl.ds(s8 * 16, 16)
            sa_v[r, sl] = sa_v[r, sl] + sb_v[r, sl]
        return carry

    lax.fori_loop(0, SRC_W, _add_src, 0)
    pltpu.async_copy(sa_v, src_out.at[pl.ds(wid * SRC_W, SRC_W)], sem_w).wait()

    def _add_nbr(r, carry):
        for s8 in range(D // 16):
            sl = pl.ds(s8 * 16, 16)
            a_v[r, sl] = a_v[r, sl] + b_v[r, sl]
        return carry

    for c in range(NCH):
        cp_a = pltpu.async_copy(mem_hbm.at[nidx_v.at[c]], a_v, sem_a)
        cp_b = pltpu.async_copy(nf_hbm.at[nidx_v.at[c]], b_v, sem_b)
        cp_a.wait()
        cp_b.wait()
        lax.fori_loop(0, CH, _add_nbr, 0)
        pltpu.async_copy(a_v, nbr_out.at[pl.ds(nbase + c * CH, CH)], sem_w).wait()


def _sc_gather(memory, node_features, src_idx, nbr_idx3):
    mesh = plsc.VectorSubcoreMesh(core_axis_name="c", subcore_axis_name="s")
    fn = pl.kernel(
        _sc_gather_body,
        mesh=mesh,
        out_type=(
            jax.ShapeDtypeStruct((B, D), jnp.float32),
            jax.ShapeDtypeStruct((BK, D), jnp.float32),
        ),
        scratch_types=[
            pltpu.VMEM((SRC_W,), jnp.int32),
            pltpu.VMEM((NCH, CH), jnp.int32),
            pltpu.VMEM((CH, D), jnp.float32),
            pltpu.VMEM((CH, D), jnp.float32),
            pltpu.VMEM((SRC_W, D), jnp.float32),
            pltpu.VMEM((SRC_W, D), jnp.float32),
            pltpu.SemaphoreType.DMA,
            pltpu.SemaphoreType.DMA,
            pltpu.SemaphoreType.DMA,
        ],
    )
    return fn(memory, node_features, src_idx, nbr_idx3)


BB = 256
_PREC = lax.Precision.DEFAULT

_INV2PI = np.float32(1.0 / (2.0 * np.pi))
_RBIG = np.float32(12582912.0)
_C1 = np.float32(6.28125)
_C2 = np.float32(2.0 * np.pi - 6.28125)
_C3 = np.float32(2.0 * np.pi - 6.28125 - float(np.float32(2.0 * np.pi - 6.28125)))
_COS_COEF = tuple(np.float32(c) for c in (
    1.0, -0.5, 0.0416666641831398, -0.0013888857793062925,
    2.4800388928269967e-05, -2.753230603502743e-07,
    2.0584800530798475e-09, -9.666989431167394e-12))


def _vcos(x):
    n = lax.round(x * _INV2PI, lax.RoundingMethod.TO_NEAREST_EVEN)
    r = ((x - n * _C1) - n * _C2) - n * _C3
    s = r * r
    acc = jnp.full_like(s, _COS_COEF[7])
    for c in _COS_COEF[6::-1]:
        acc = acc * s + c
    return acc


def _tc_body(src_ref, nbr_ref, ef_ref, ts_ref, ets_ref, nid_ref, tw_ref, tb_ref,
             wq1, wq2, wk1, wk2, wk3, wv1, wv2, wv3, wo1, wo2, out_ref):
    src = src_ref[...]
    tw = tw_ref[...]
    tb = tb_ref[...]
    q_const = jnp.dot(_vcos(tb), wq2[...], precision=_PREC)
    q = jnp.dot(src, wq1[...], precision=_PREC) + q_const
    scale = np.float32(1.0 / np.sqrt(DH))

    neg = jnp.full((BB, 1), -1e30, dtype=jnp.float32)
    m0, m1 = neg, neg
    l0 = jnp.zeros((BB, 1), dtype=jnp.float32)
    l1 = jnp.zeros((BB, 1), dtype=jnp.float32)
    acc0 = jnp.zeros((BB, DH), dtype=jnp.float32)
    acc1 = jnp.zeros((BB, DH), dtype=jnp.float32)

    for k in range(K):
        nbr_k = nbr_ref[k]
        ef_k = ef_ref[:, k * D_EDGE:(k + 1) * D_EDGE]
        delta = ts_ref[...] - ets_ref[:, k:k + 1]
        te = _vcos(delta * tw + tb)
        kk = (jnp.dot(nbr_k, wk1[...], precision=_PREC)
              + jnp.dot(te, wk2[...], precision=_PREC)
              + jnp.dot(ef_k, wk3[...], precision=_PREC))
        vv = (jnp.dot(nbr_k, wv1[...], precision=_PREC)
              + jnp.dot(te, wv2[...], precision=_PREC)
              + jnp.dot(ef_k, wv3[...], precision=_PREC))
        prod = q * kk
        s0 = jnp.sum(prod[:, :DH], axis=1, keepdims=True) * scale
        s1 = jnp.sum(prod[:, DH:], axis=1, keepdims=True) * scale
        is_pad = nid_ref[:, k:k + 1] == 0
        s0 = jnp.where(is_pad, jnp.float32(-1e9), s0)
        s1 = jnp.where(is_pad, jnp.float32(-1e9), s1)
        m0n = jnp.maximum(m0, s0)
        m1n = jnp.maximum(m1, s1)
        a0 = jnp.exp(m0 - m0n)
        a1 = jnp.exp(m1 - m1n)
        p0 = jnp.exp(s0 - m0n)
        p1 = jnp.exp(s1 - m1n)
        l0 = l0 * a0 + p0
        l1 = l1 * a1 + p1
        acc0 = acc0 * a0 + p0 * vv[:, :DH]
        acc1 = acc1 * a1 + p1 * vv[:, DH:]
        m0, m1 = m0n, m1n

    attn = jnp.concatenate([acc0 / l0, acc1 / l1], axis=1)
    out_ref[...] = (jnp.dot(attn, wo1[...], precision=_PREC)
                    + jnp.dot(src, wo2[...], precision=_PREC))


def _tc_dense(src_conv, nbr3, ef2, ts2, ets, nids, time_w, time_b2,
              Wq1, Wq2, Wk1, Wk2, Wk3, Wv1, Wv2, Wv3, Wo1, Wo2):
    row = lambda i: (i, 0)
    fixed = lambda i: (0, 0)
    return pl.pallas_call(
        _tc_body,
        grid=(B // BB,),
        in_specs=[
            pl.BlockSpec((BB, D), row),
            pl.BlockSpec((K, BB, D), lambda i: (0, i, 0)),
            pl.BlockSpec((BB, K * D_EDGE), row),
            pl.BlockSpec((BB, 1), row),
            pl.BlockSpec((BB, K), row),
            pl.BlockSpec((BB, K), row),
            pl.BlockSpec((1, D_TIME), fixed),
            pl.BlockSpec((1, D_TIME), fixed),
            pl.BlockSpec((D, D), fixed),
            pl.BlockSpec((D_TIME, D), fixed),
            pl.BlockSpec((D, D), fixed),
            pl.BlockSpec((D_TIME, D), fixed),
            pl.BlockSpec((D_EDGE, D), fixed),
            pl.BlockSpec((D, D), fixed),
            pl.BlockSpec((D_TIME, D), fixed),
            pl.BlockSpec((D_EDGE, D), fixed),
            pl.BlockSpec((D, D), fixed),
            pl.BlockSpec((D, D), fixed),
        ],
        out_specs=pl.BlockSpec((BB, D), row),
        out_shape=jax.ShapeDtypeStruct((B, D), jnp.float32),
    )(src_conv, nbr3, ef2, ts2, ets, nids, time_w, time_b2,
      Wq1, Wq2, Wk1, Wk2, Wk3, Wv1, Wv2, Wv3, Wo1, Wo2)


def kernel(memory, source_nodes, timestamps, n_layers, neighbors, edge_idxs,
           edge_times, node_features, edge_features, time_w, time_b,
           Wq, Wk, Wv, Wout):
    del n_layers
    src_idx = source_nodes.astype(jnp.int32)
    nbr_idx3 = neighbors.astype(jnp.int32).T.reshape(NW, NCH, CH)

    src_conv, nbr_emb = _sc_gather(memory, node_features, src_idx, nbr_idx3)
    nbr3 = nbr_emb.reshape(K, B, D)

    ef2 = jnp.take(edge_features, edge_idxs.reshape(-1), axis=0).reshape(B, K * D_EDGE)

    ts2 = timestamps.reshape(B, 1)
    time_b2 = time_b.reshape(1, D_TIME)

    Wq1, Wq2 = Wq[:D], Wq[D:]
    Wk1, Wk2, Wk3 = Wk[:D], Wk[D:D + D_TIME], Wk[D + D_TIME:]
    Wv1, Wv2, Wv3 = Wv[:D], Wv[D:D + D_TIME], Wv[D + D_TIME:]
    Wo1, Wo2 = Wout[:D], Wout[D:]

    return _tc_dense(src_conv, nbr3, ef2, ts2, edge_times, neighbors.astype(jnp.int32),
                     time_w, time_b2, Wq1, Wq2, Wk1, Wk2, Wk3, Wv1, Wv2, Wv3, Wo1, Wo2)
